# baseline (speedup 1.0000x reference)
import functools

import jax
import jax.numpy as jnp
from jax import lax
from jax.experimental import pallas as pl
from jax.experimental.pallas import tpu as pltpu
from jax.experimental.pallas import tpu_sc as plsc

VOCAB = 4
NSEG = 2
SEQ = 30
D = 768
BATCH = 4096
NTOK = BATCH * SEQ
NJ = VOCAB * NSEG
NROWS = SEQ * NJ

BB = 128
TB = BB * SEQ


def _comb_body(tok_ref, pos_ref, seg_ref, out_ref):
    tok8 = jnp.concatenate(
        [tok_ref[v : v + 1] for v in range(VOCAB) for _ in range(NSEG)], axis=0
    )
    seg8 = jnp.concatenate(
        [seg_ref[g : g + 1] for _ in range(VOCAB) for g in range(NSEG)], axis=0
    )
    base = tok8 + seg8
    for s in range(SEQ):
        e = base + pos_ref[s : s + 1]
        m = jnp.mean(e, axis=-1, keepdims=True)
        var = jnp.mean((e - m) ** 2, axis=-1, keepdims=True)
        out_ref[s * NJ : (s + 1) * NJ] = ((e - m) * lax.rsqrt(var + 1e-5)).astype(
            jnp.bfloat16
        )


def _build_comb(tok_table, pos_table, seg_table):
    return pl.pallas_call(
        _comb_body,
        out_shape=jax.ShapeDtypeStruct((NROWS, D), jnp.bfloat16),
    )(tok_table, pos_table, seg_table)


def _expand_body(x_ref, s_ref, comb_ref, out_ref):
    xb = x_ref[...]
    sb = s_ref[...]
    pos = lax.broadcasted_iota(jnp.int32, (TB, 1), 0) % SEQ
    c = pos * NJ + xb * NSEG + sb
    oh = (c == lax.broadcasted_iota(jnp.int32, (TB, NROWS), 1)).astype(
        jnp.bfloat16
    )
    res = lax.dot_general(
        oh,
        comb_ref[...],
        (((1,), (0,)), ((), ())),
        preferred_element_type=jnp.float32,
    )
    out_ref[...] = res.reshape(BB, SEQ, D)


def _expand(xi, si, comb):
    return pl.pallas_call(
        _expand_body,
        grid=(BATCH // BB,),
        in_specs=[
            pl.BlockSpec((TB, 1), lambda b: (b, 0)),
            pl.BlockSpec((TB, 1), lambda b: (b, 0)),
            pl.BlockSpec((NROWS, D), lambda b: (0, 0)),
        ],
        out_specs=pl.BlockSpec((BB, SEQ, D), lambda b: (b, 0, 0)),
        out_shape=jax.ShapeDtypeStruct((BATCH, SEQ, D), jnp.float32),
    )(xi, si, comb)


def kernel(x, seg, tok_table, pos_table, seg_table):
    comb = _build_comb(tok_table, pos_table, seg_table)
    xi = x.astype(jnp.int32).reshape(NTOK, 1)
    si = seg.astype(jnp.int32).reshape(NTOK, 1)
    return _expand(xi, si, comb)


# speedup vs baseline: 1.0650x; 1.0650x over previous
import functools

import jax
import jax.numpy as jnp
from jax import lax
from jax.experimental import pallas as pl
from jax.experimental.pallas import tpu as pltpu
from jax.experimental.pallas import tpu_sc as plsc

VOCAB = 4
NSEG = 2
SEQ = 30
D = 768
BATCH = 4096
NTOK = BATCH * SEQ
NJ = VOCAB * NSEG
NROWS = SEQ * NJ

BB = 128
TB = BB * SEQ


def _comb_body(tok_ref, pos_ref, seg_ref, out_ref):
    tok8 = jnp.concatenate(
        [tok_ref[v : v + 1] for v in range(VOCAB) for _ in range(NSEG)], axis=0
    )
    seg8 = jnp.concatenate(
        [seg_ref[g : g + 1] for _ in range(VOCAB) for g in range(NSEG)], axis=0
    )
    base = tok8 + seg8
    for s in range(SEQ):
        e = base + pos_ref[s : s + 1]
        m = jnp.mean(e, axis=-1, keepdims=True)
        var = jnp.mean((e - m) ** 2, axis=-1, keepdims=True)
        out_ref[s * NJ : (s + 1) * NJ] = ((e - m) * lax.rsqrt(var + 1e-5)).astype(
            jnp.bfloat16
        )


def _build_comb(tok_table, pos_table, seg_table):
    return pl.pallas_call(
        _comb_body,
        out_shape=jax.ShapeDtypeStruct((NROWS, D), jnp.bfloat16),
    )(tok_table, pos_table, seg_table)


def _expand_body(x_ref, s_ref, comb_ref, out_ref):
    xb = x_ref[...].astype(jnp.int32)
    sb = s_ref[...].astype(jnp.int32)
    pos = lax.broadcasted_iota(jnp.int32, (TB, 1), 0) % SEQ
    c = pos * NJ + xb * NSEG + sb
    oh = (c == lax.broadcasted_iota(jnp.int32, (TB, NROWS), 1)).astype(
        jnp.bfloat16
    )
    res = lax.dot_general(
        oh,
        comb_ref[...],
        (((1,), (0,)), ((), ())),
        preferred_element_type=jnp.float32,
    )
    out_ref[...] = res.reshape(BB, SEQ, D)


def _expand(xi, si, comb):
    return pl.pallas_call(
        _expand_body,
        grid=(BATCH // BB,),
        in_specs=[
            pl.BlockSpec((TB, 1), lambda b: (b, 0)),
            pl.BlockSpec((TB, 1), lambda b: (b, 0)),
            pl.BlockSpec((NROWS, D), lambda b: (0, 0)),
        ],
        out_specs=pl.BlockSpec((BB, SEQ, D), lambda b: (b, 0, 0)),
        out_shape=jax.ShapeDtypeStruct((BATCH, SEQ, D), jnp.float32),
    )(xi, si, comb)


def kernel(x, seg, tok_table, pos_table, seg_table):
    comb = _build_comb(tok_table, pos_table, seg_table)
    xi = x.astype(jnp.int8).reshape(NTOK, 1)
    si = seg.astype(jnp.int8).reshape(NTOK, 1)
    return _expand(xi, si, comb)
